# baseline (speedup 1.0000x reference)
import functools

import jax
import jax.numpy as jnp
from jax.experimental import pallas as pl
from jax.experimental.pallas import tpu as pltpu

_NBUF = 4


def _route_row(xh, w, bias, eiota):
    logits = jnp.dot(w, xh, preferred_element_type=jnp.float32) + bias
    E = logits.shape[0]
    m1 = jnp.max(logits, axis=0, keepdims=True)
    i1 = jnp.min(jnp.where(logits == m1, eiota, E), axis=0, keepdims=True)
    masked = jnp.where(eiota == i1, -jnp.inf, logits)
    m2 = jnp.max(masked, axis=0, keepdims=True)
    i2 = jnp.min(jnp.where(masked == m2, eiota, E), axis=0, keepdims=True)
    ex = jnp.exp(logits - m1)
    recip = 1.0 / jnp.sum(ex, axis=0, keepdims=True)
    v1 = recip
    v2 = jnp.exp(m2 - m1) * recip
    return i1, i2, v1, v2


def _make_body(H, Hb, n_hi):
    def body(x_ref, w_ref, b_ref, gates_ref, idx_ref, val_ref, xh_buf, sem):
        w = w_ref[...]
        bias = b_ref[...]
        E = w.shape[0]
        Wd = x_ref.shape[3]
        eiota = jax.lax.broadcasted_iota(jnp.int32, (E, Wd), 0)

        bi = pl.program_id(0)
        hi = pl.program_id(1)
        step = bi * n_hi + hi
        base = step * Hb
        n_steps = pl.num_programs(0) * n_hi

        def copy_row(g, slot):
            gb = g // H
            gh = g % H
            return pltpu.make_async_copy(
                x_ref.at[gb, :, gh, :], xh_buf.at[slot], sem.at[slot])

        @pl.when(step == 0)
        def _():
            for k in range(_NBUF - 1):
                copy_row(base + k, k % _NBUF).start()

        i1s, i2s, v1s, v2s = [], [], [], []
        for h in range(Hb):
            g = base + h
            nxt = g + _NBUF - 1
            n_rows = n_steps * Hb

            @pl.when(nxt < n_rows)
            def _():
                copy_row(nxt, nxt % _NBUF).start()
            copy_row(g, g % _NBUF).wait()
            i1, i2, v1, v2 = _route_row(xh_buf[g % _NBUF], w, bias, eiota)
            i1s.append(i1)
            i2s.append(i2)
            v1s.append(v1)
            v2s.append(v2)

        I1 = jnp.concatenate(i1s, axis=0)
        I2 = jnp.concatenate(i2s, axis=0)
        V1 = jnp.concatenate(v1s, axis=0)
        V2 = jnp.concatenate(v2s, axis=0)
        idx_ref[0, 0] = I1
        idx_ref[0, 1] = I2
        val_ref[0, 0] = V1
        val_ref[0, 1] = V2
        zero = jnp.zeros_like(V1)
        for e in range(E):
            gates_ref[0, e] = (jnp.where(I1 == e, V1, zero)
                               + jnp.where(I2 == e, V2, zero))
    return body


@functools.partial(jax.jit, static_argnames=())
def kernel(x, W, b):
    B, C, H, Wd = x.shape
    E = W.shape[0]
    b2 = b.reshape(E, 1)
    Hb = 8 if H % 8 == 0 else 1
    n_hi = H // Hb
    grid = (B, n_hi)

    gates, idx, vals = pl.pallas_call(
        _make_body(H, Hb, n_hi),
        grid=grid,
        in_specs=[
            pl.BlockSpec(memory_space=pltpu.MemorySpace.HBM),
            pl.BlockSpec((E, C), lambda bi, hi: (0, 0)),
            pl.BlockSpec((E, 1), lambda bi, hi: (0, 0)),
        ],
        out_specs=[
            pl.BlockSpec((1, E, Hb, Wd), lambda bi, hi: (bi, 0, hi, 0)),
            pl.BlockSpec((1, 2, Hb, Wd), lambda bi, hi: (bi, 0, hi, 0)),
            pl.BlockSpec((1, 2, Hb, Wd), lambda bi, hi: (bi, 0, hi, 0)),
        ],
        out_shape=[
            jax.ShapeDtypeStruct((B, E, H, Wd), jnp.float32),
            jax.ShapeDtypeStruct((B, 2, H, Wd), jnp.int32),
            jax.ShapeDtypeStruct((B, 2, H, Wd), jnp.float32),
        ],
        scratch_shapes=[
            pltpu.VMEM((_NBUF, C, Wd), jnp.float32),
            pltpu.SemaphoreType.DMA((_NBUF,)),
        ],
    )(x, W, b2)

    return gates, idx, vals


# speedup vs baseline: 1.1325x; 1.1325x over previous
import functools

import jax
import jax.numpy as jnp
from jax.experimental import pallas as pl
from jax.experimental.pallas import tpu as pltpu

_NBUF = 8


def _route_row(xh, w, bias, eiota):
    logits = jnp.dot(w, xh, preferred_element_type=jnp.float32) + bias
    E = logits.shape[0]
    m1 = jnp.max(logits, axis=0, keepdims=True)
    i1 = jnp.min(jnp.where(logits == m1, eiota, E), axis=0, keepdims=True)
    masked = jnp.where(eiota == i1, -jnp.inf, logits)
    m2 = jnp.max(masked, axis=0, keepdims=True)
    i2 = jnp.min(jnp.where(masked == m2, eiota, E), axis=0, keepdims=True)
    ex = jnp.exp(logits - m1)
    recip = 1.0 / jnp.sum(ex, axis=0, keepdims=True)
    v1 = recip
    v2 = jnp.exp(m2 - m1) * recip
    return i1, i2, v1, v2


def _make_body(H, Hb, n_hi):
    def body(x_ref, w_ref, b_ref, gates_ref, idx_ref, val_ref, xh_buf, sem):
        w = w_ref[...]
        bias = b_ref[...]
        E = w.shape[0]
        Wd = x_ref.shape[3]
        eiota = jax.lax.broadcasted_iota(jnp.int32, (E, Wd), 0)

        bi = pl.program_id(0)
        hi = pl.program_id(1)
        step = bi * n_hi + hi
        base = step * Hb
        n_steps = pl.num_programs(0) * n_hi

        def copy_row(g, slot):
            gb = g // H
            gh = g % H
            return pltpu.make_async_copy(
                x_ref.at[gb, :, gh, :], xh_buf.at[slot], sem.at[slot])

        @pl.when(step == 0)
        def _():
            for k in range(_NBUF - 1):
                copy_row(base + k, k % _NBUF).start()

        i1s, i2s, v1s, v2s = [], [], [], []
        for h in range(Hb):
            g = base + h
            nxt = g + _NBUF - 1
            n_rows = n_steps * Hb

            @pl.when(nxt < n_rows)
            def _():
                copy_row(nxt, nxt % _NBUF).start()
            copy_row(g, g % _NBUF).wait()
            i1, i2, v1, v2 = _route_row(xh_buf[g % _NBUF], w, bias, eiota)
            i1s.append(i1)
            i2s.append(i2)
            v1s.append(v1)
            v2s.append(v2)

        I1 = jnp.concatenate(i1s, axis=0)
        I2 = jnp.concatenate(i2s, axis=0)
        V1 = jnp.concatenate(v1s, axis=0)
        V2 = jnp.concatenate(v2s, axis=0)
        idx_ref[0, 0] = I1
        idx_ref[0, 1] = I2
        val_ref[0, 0] = V1
        val_ref[0, 1] = V2
        zero = jnp.zeros_like(V1)
        for e in range(E):
            gates_ref[0, e] = (jnp.where(I1 == e, V1, zero)
                               + jnp.where(I2 == e, V2, zero))
    return body


@functools.partial(jax.jit, static_argnames=())
def kernel(x, W, b):
    B, C, H, Wd = x.shape
    E = W.shape[0]
    b2 = b.reshape(E, 1)
    Hb = 8 if H % 8 == 0 else 1
    n_hi = H // Hb
    grid = (B, n_hi)

    gates, idx, vals = pl.pallas_call(
        _make_body(H, Hb, n_hi),
        grid=grid,
        in_specs=[
            pl.BlockSpec(memory_space=pltpu.MemorySpace.HBM),
            pl.BlockSpec((E, C), lambda bi, hi: (0, 0)),
            pl.BlockSpec((E, 1), lambda bi, hi: (0, 0)),
        ],
        out_specs=[
            pl.BlockSpec((1, E, Hb, Wd), lambda bi, hi: (bi, 0, hi, 0)),
            pl.BlockSpec((1, 2, Hb, Wd), lambda bi, hi: (bi, 0, hi, 0)),
            pl.BlockSpec((1, 2, Hb, Wd), lambda bi, hi: (bi, 0, hi, 0)),
        ],
        out_shape=[
            jax.ShapeDtypeStruct((B, E, H, Wd), jnp.float32),
            jax.ShapeDtypeStruct((B, 2, H, Wd), jnp.int32),
            jax.ShapeDtypeStruct((B, 2, H, Wd), jnp.float32),
        ],
        scratch_shapes=[
            pltpu.VMEM((_NBUF, C, Wd), jnp.float32),
            pltpu.SemaphoreType.DMA((_NBUF,)),
        ],
    )(x, W, b2)

    return gates, idx, vals


# speedup vs baseline: 1.5443x; 1.3636x over previous
import functools

import jax
import jax.numpy as jnp
from jax.experimental import pallas as pl


def _to_rows(v, hb, wd):
    return jnp.concatenate([v[:, j * wd:(j + 1) * wd] for j in range(hb)],
                           axis=0)


def _routing_body(x_ref, w_ref, b_ref, gates_ref, idx_ref, val_ref):
    w = w_ref[...]
    bias = b_ref[...]
    E = w.shape[0]
    Hb = gates_ref.shape[2]
    Wd = gates_ref.shape[3]
    xb = x_ref[0]

    logits = jnp.dot(w, xb, preferred_element_type=jnp.float32) + bias
    T = logits.shape[1]
    eiota = jax.lax.broadcasted_iota(jnp.int32, (E, T), 0)
    m1 = jnp.max(logits, axis=0, keepdims=True)
    i1 = jnp.min(jnp.where(logits == m1, eiota, E), axis=0, keepdims=True)
    masked = jnp.where(eiota == i1, -jnp.inf, logits)
    m2 = jnp.max(masked, axis=0, keepdims=True)
    i2 = jnp.min(jnp.where(masked == m2, eiota, E), axis=0, keepdims=True)
    ex = jnp.exp(logits - m1)
    recip = 1.0 / jnp.sum(ex, axis=0, keepdims=True)
    v1 = recip
    v2 = jnp.exp(m2 - m1) * recip

    I1 = _to_rows(i1, Hb, Wd)
    I2 = _to_rows(i2, Hb, Wd)
    V1 = _to_rows(v1, Hb, Wd)
    V2 = _to_rows(v2, Hb, Wd)
    idx_ref[0, 0] = I1
    idx_ref[0, 1] = I2
    val_ref[0, 0] = V1
    val_ref[0, 1] = V2
    zero = jnp.zeros_like(V1)
    for e in range(E):
        gates_ref[0, e] = (jnp.where(I1 == e, V1, zero)
                           + jnp.where(I2 == e, V2, zero))


@functools.partial(jax.jit, static_argnames=())
def kernel(x, W, b):
    B, C, H, Wd = x.shape
    E = W.shape[0]
    S = H * Wd
    xr = x.reshape(B, C, S)
    b2 = b.reshape(E, 1)
    Hb = 8 if H % 8 == 0 else 1
    T = Hb * Wd
    grid = (B, H // Hb)

    gates, idx, vals = pl.pallas_call(
        _routing_body,
        grid=grid,
        in_specs=[
            pl.BlockSpec((1, C, T), lambda bi, hi: (bi, 0, hi)),
            pl.BlockSpec((E, C), lambda bi, hi: (0, 0)),
            pl.BlockSpec((E, 1), lambda bi, hi: (0, 0)),
        ],
        out_specs=[
            pl.BlockSpec((1, E, Hb, Wd), lambda bi, hi: (bi, 0, hi, 0)),
            pl.BlockSpec((1, 2, Hb, Wd), lambda bi, hi: (bi, 0, hi, 0)),
            pl.BlockSpec((1, 2, Hb, Wd), lambda bi, hi: (bi, 0, hi, 0)),
        ],
        out_shape=[
            jax.ShapeDtypeStruct((B, E, H, Wd), jnp.float32),
            jax.ShapeDtypeStruct((B, 2, H, Wd), jnp.int32),
            jax.ShapeDtypeStruct((B, 2, H, Wd), jnp.float32),
        ],
    )(xr, W, b2)

    return gates, idx, vals
